# baseline (speedup 1.0000x reference)
import jax
import jax.numpy as jnp
from jax import lax
from jax.experimental import pallas as pl
from jax.experimental.pallas import tpu as pltpu
from jax.experimental.pallas import tpu_sc as plsc

N = 10000
E = 160000
D_IN = 128
D_H = 256
H = 128
EPS = 1e-5
BLK = 1000
GRID = N // BLK

NS = 16
NC = 2
EP = E // NS
CH = 80
NSEC = 5
SECE = EP // NSEC
SCH = SECE // CH
NZ = 640
NZL = N - (NS - 1) * NZ
ZR = 40


def _mp_body(h2, srcg, dstg, ea, agg2, idx2d, dst2d, ea_sec, b0, b1, b2,
             acc, g0, g1, g2, s0, s1, s2):
    bufs = (b0, b1, b2)
    gsems = (g0, g1, g2)
    ssems = (s0, s1, s2)
    c = lax.axis_index("c")
    s = lax.axis_index("s")
    cN = c * N
    n0 = s * NZ

    zv = jnp.zeros((16,), jnp.float32)
    for r in range(ZR):
        for q in range(H // 16):
            bufs[0][r, pl.ds(q * 16, 16)] = zv

    @pl.when(s < NS - 1)
    def _():
        zds = [pltpu.async_copy(bufs[0].at[pl.ds(0, ZR)],
                                acc.at[pl.ds(n0 + i * ZR, ZR)], gsems[0])
               for i in range(NZ // ZR)]
        for d in zds:
            d.wait()

    @pl.when(s == NS - 1)
    def _():
        zds = [pltpu.async_copy(bufs[0].at[pl.ds(0, ZR)],
                                acc.at[pl.ds(n0 + i * ZR, ZR)], gsems[0])
               for i in range(NZL // ZR)]
        for d in zds:
            d.wait()

    plsc.subcore_barrier()

    def g_start(k, x):
        pltpu.async_copy(h2.at[idx2d.at[k]], bufs[x], gsems[x])

    def g_wait(x):
        pltpu.make_async_copy(h2.at[idx2d.at[0]], bufs[x], gsems[x]).wait()

    def s_start(k, x):
        pltpu.async_copy(bufs[x], acc.at[dst2d.at[k]], ssems[x], add=True)

    def s_wait(x):
        pltpu.make_async_copy(bufs[x], acc.at[dst2d.at[0]], ssems[x]).wait()

    def scale(k, x):
        buf = bufs[x]

        @plsc.parallel_loop(0, CH // 16)
        def grp(t):
            ea16 = ea_sec[pl.ds(k * CH + t * 16, 16)]
            for j in range(16):
                w = ea16[j]
                e = t * 16 + j
                for q in range(H // 16):
                    buf[e, pl.ds(q * 16, 16)] = (
                        buf[e, pl.ds(q * 16, 16)] * w)

    def section(sec, carry):
        pltpu.sync_copy(srcg.at[s, sec], idx2d)
        pltpu.sync_copy(dstg.at[s, sec], dst2d)
        pltpu.sync_copy(ea.at[pl.ds(s * EP + sec * SECE, SECE)], ea_sec)

        def addcn(r, carry2):
            for q in range(CH // 16):
                idx2d[r, pl.ds(q * 16, 16)] = (
                    idx2d[r, pl.ds(q * 16, 16)] + cN)
            return carry2
        lax.fori_loop(0, SCH, addcn, 0)

        g_start(0, 0)
        g_start(1, 1)

        def triple(k3, carry2):
            k = 3 * k3
            g_wait(0)
            scale(k, 0)
            s_start(k, 0)

            @pl.when(k3 > 0)
            def _():
                s_wait(2)
            g_start(k + 2, 2)
            g_wait(1)
            scale(k + 1, 1)
            s_start(k + 1, 1)
            s_wait(0)
            g_start(k + 3, 0)
            g_wait(2)
            scale(k + 2, 2)
            s_start(k + 2, 2)
            s_wait(1)

            @pl.when(k3 < (SCH - 1) // 3 - 1)
            def _():
                g_start(k + 4, 1)
            return carry2

        lax.fori_loop(0, (SCH - 1) // 3, triple, 0)

        g_wait(0)
        scale(SCH - 1, 0)
        s_start(SCH - 1, 0)
        s_wait(2)
        s_wait(0)
        return carry

    lax.fori_loop(0, NSEC, section, 0)

    plsc.subcore_barrier()

    @pl.when(s < NS - 1)
    def _():
        pltpu.sync_copy(acc.at[pl.ds(n0, NZ)],
                        agg2.at[pl.ds(cN + n0, NZ)])

    @pl.when(s == NS - 1)
    def _():
        pltpu.sync_copy(acc.at[pl.ds(n0, NZL)],
                        agg2.at[pl.ds(cN + n0, NZL)])


_mp = pl.kernel(
    _mp_body,
    out_type=jax.ShapeDtypeStruct((NC * N, H), jnp.float32),
    mesh=plsc.VectorSubcoreMesh(core_axis_name="c", subcore_axis_name="s"),
    scratch_types=[
        pltpu.VMEM((SCH, CH), jnp.int32),
        pltpu.VMEM((SCH, CH), jnp.int32),
        pltpu.VMEM((SECE,), jnp.float32),
        pltpu.VMEM((CH, H), jnp.float32),
        pltpu.VMEM((CH, H), jnp.float32),
        pltpu.VMEM((CH, H), jnp.float32),
        pltpu.VMEM_SHARED((N, H), jnp.float32),
        pltpu.SemaphoreType.DMA,
        pltpu.SemaphoreType.DMA,
        pltpu.SemaphoreType.DMA,
        pltpu.SemaphoreType.DMA,
        pltpu.SemaphoreType.DMA,
        pltpu.SemaphoreType.DMA,
    ],
)


def _proj_body(x_ref, w_ref, b_ref, out_ref):
    h = jnp.dot(x_ref[...], w_ref[...], preferred_element_type=jnp.float32)
    h = jnp.maximum(h + b_ref[...], 0.0)
    out_ref[0] = h[:, :H]
    out_ref[1] = h[:, H:]


_proj = pl.pallas_call(
    _proj_body,
    grid=(GRID,),
    in_specs=[
        pl.BlockSpec((BLK, D_IN), lambda i: (i, 0)),
        pl.BlockSpec((D_IN, D_H), lambda i: (0, 0)),
        pl.BlockSpec((1, D_H), lambda i: (0, 0)),
    ],
    out_specs=pl.BlockSpec((2, BLK, H), lambda i: (0, i, 0)),
    out_shape=jax.ShapeDtypeStruct((2, N, H), jnp.float32),
)


def _mm_body(a0_ref, a1_ref, wt_ref, wb_ref, b_ref, out_ref, st_ref):
    i = pl.program_id(0)
    o = (jnp.dot(a0_ref[0], wt_ref[...], preferred_element_type=jnp.float32)
         + jnp.dot(a1_ref[0], wb_ref[...], preferred_element_type=jnp.float32)
         + b_ref[...])
    out_ref[...] = o
    st = jnp.concatenate([jnp.sum(o, axis=0)[None],
                          jnp.sum(o * o, axis=0)[None]], axis=0)

    @pl.when(i == 0)
    def _():
        st_ref[...] = st

    @pl.when(i > 0)
    def _():
        st_ref[...] = st_ref[...] + st


_mm_stats = pl.pallas_call(
    _mm_body,
    grid=(GRID,),
    in_specs=[
        pl.BlockSpec((1, BLK, H), lambda i: (0, i, 0)),
        pl.BlockSpec((1, BLK, H), lambda i: (1, i, 0)),
        pl.BlockSpec((H, D_H), lambda i: (0, 0)),
        pl.BlockSpec((H, D_H), lambda i: (0, 0)),
        pl.BlockSpec((1, D_H), lambda i: (0, 0)),
    ],
    out_specs=[
        pl.BlockSpec((BLK, D_H), lambda i: (i, 0)),
        pl.BlockSpec((2, D_H), lambda i: (0, 0)),
    ],
    out_shape=[
        jax.ShapeDtypeStruct((N, D_H), jnp.float32),
        jax.ShapeDtypeStruct((2, D_H), jnp.float32),
    ],
)


def _nr_body(o_ref, st_ref, g_ref, be_ref, h_ref, hn_ref, sm_ref):
    i = pl.program_id(0)
    inv_cnt = 1.0 / (N * D_H)
    mu = jnp.sum(st_ref[0:1, :]) * inv_cnt
    ex2 = jnp.sum(st_ref[1:2, :]) * inv_cnt
    inv = lax.rsqrt(ex2 - mu * mu + EPS)
    nrm = (o_ref[...] - mu) * inv * g_ref[...] + be_ref[...]
    h0 = jnp.maximum(nrm[:, :H] + h_ref[0], 0.0)
    h1 = jnp.maximum(nrm[:, H:] + h_ref[1], 0.0)
    hn_ref[0] = h0
    hn_ref[1] = h1
    sm = jnp.concatenate([jnp.sum(h0, axis=0)[None],
                          jnp.sum(h1, axis=0)[None]], axis=0)

    @pl.when(i == 0)
    def _():
        sm_ref[...] = sm

    @pl.when(i > 0)
    def _():
        sm_ref[...] = sm_ref[...] + sm


_norm_resid = pl.pallas_call(
    _nr_body,
    grid=(GRID,),
    in_specs=[
        pl.BlockSpec((BLK, D_H), lambda i: (i, 0)),
        pl.BlockSpec((2, D_H), lambda i: (0, 0)),
        pl.BlockSpec((1, D_H), lambda i: (0, 0)),
        pl.BlockSpec((1, D_H), lambda i: (0, 0)),
        pl.BlockSpec((2, BLK, H), lambda i: (0, i, 0)),
    ],
    out_specs=[
        pl.BlockSpec((2, BLK, H), lambda i: (0, i, 0)),
        pl.BlockSpec((2, H), lambda i: (0, 0)),
    ],
    out_shape=[
        jax.ShapeDtypeStruct((2, N, H), jnp.float32),
        jax.ShapeDtypeStruct((2, H), jnp.float32),
    ],
)


def _pool_body(sm_ref, wp_ref, bp_ref, out_ref):
    p = jnp.concatenate([sm_ref[0:1, :], sm_ref[1:2, :]], axis=1) * (1.0 / N)
    p8 = jnp.broadcast_to(p, (8, D_H))
    o = jnp.dot(p8, wp_ref[...], preferred_element_type=jnp.float32)
    o = jnp.maximum(o + bp_ref[...], 0.0)
    out_ref[...] = o[0:1, :]


_pool = pl.pallas_call(
    _pool_body,
    in_specs=[
        pl.BlockSpec((2, H), lambda: (0, 0)),
        pl.BlockSpec((D_H, D_H), lambda: (0, 0)),
        pl.BlockSpec((1, D_H), lambda: (0, 0)),
    ],
    out_specs=pl.BlockSpec((1, D_H), lambda: (0, 0)),
    out_shape=jax.ShapeDtypeStruct((1, D_H), jnp.float32),
)


def kernel(x, edge_index, edge_attr, W_in, b_in, W0, b0, g0, be0,
           W1, b1, g1, be1, W2, b2, g2, be2, W_pool, b_pool):
    srcg = edge_index[0].astype(jnp.int32).reshape(NS, NSEC, SCH, CH)
    dstg = edge_index[1].astype(jnp.int32).reshape(NS, NSEC, SCH, CH)
    ea = edge_attr.astype(jnp.float32)

    h = _proj(x, W_in, b_in.reshape(1, D_H))
    sums = None
    for (W, b, g, be) in ((W0, b0, g0, be0), (W1, b1, g1, be1),
                          (W2, b2, g2, be2)):
        agg2 = _mp(h.reshape(NC * N, H), srcg, dstg, ea)
        agg = agg2.reshape(NC, N, H)
        out, stats = _mm_stats(agg, agg, W[:H, :], W[H:, :],
                               b.reshape(1, D_H))
        h, sums = _norm_resid(out, stats, g.reshape(1, D_H),
                              be.reshape(1, D_H), h)
    return _pool(sums, W_pool, b_pool.reshape(1, D_H))


# speedup vs baseline: 1.1506x; 1.1506x over previous
import jax
import jax.numpy as jnp
from jax import lax
from jax.experimental import pallas as pl
from jax.experimental.pallas import tpu as pltpu
from jax.experimental.pallas import tpu_sc as plsc

N = 10000
E = 160000
D_IN = 128
D_H = 256
H = 128
EPS = 1e-5
BLK = 1000
GRID = N // BLK

NS = 16
NC = 2
EP = E // NS
CH = 80
NSEC = 5
SECE = EP // NSEC
SCH = SECE // CH
NZ = 640
NZL = N - (NS - 1) * NZ
ZR = 40


def _mp_body(h2, srcg, dstg, ea, agg2, idx2d, dst2d, ea_sec, b0, b1, b2,
             acc, g0, g1, g2, s0, s1, s2):
    bufs = (b0, b1, b2)
    gsems = (g0, g1, g2)
    ssems = (s0, s1, s2)
    c = lax.axis_index("c")
    s = lax.axis_index("s")
    cN = c * N
    n0 = s * NZ

    zv = jnp.zeros((16,), jnp.float32)
    for r in range(ZR):
        for q in range(H // 16):
            bufs[0][r, pl.ds(q * 16, 16)] = zv

    @pl.when(s < NS - 1)
    def _():
        zds = [pltpu.async_copy(bufs[0].at[pl.ds(0, ZR)],
                                acc.at[pl.ds(n0 + i * ZR, ZR)], gsems[0])
               for i in range(NZ // ZR)]
        for d in zds:
            d.wait()

    @pl.when(s == NS - 1)
    def _():
        zds = [pltpu.async_copy(bufs[0].at[pl.ds(0, ZR)],
                                acc.at[pl.ds(n0 + i * ZR, ZR)], gsems[0])
               for i in range(NZL // ZR)]
        for d in zds:
            d.wait()

    plsc.subcore_barrier()

    def g_start(k, x):
        pltpu.async_copy(h2.at[idx2d.at[k]], bufs[x], gsems[x])

    def g_wait(x):
        pltpu.make_async_copy(h2.at[idx2d.at[0]], bufs[x], gsems[x]).wait()

    def s_start(k, x):
        pltpu.async_copy(bufs[x], acc.at[dst2d.at[k]], ssems[x], add=True)

    def s_wait(x):
        pltpu.make_async_copy(bufs[x], acc.at[dst2d.at[0]], ssems[x]).wait()

    def scale(k, x):
        buf = bufs[x]

        def grp(t, carry2):
            ea16 = ea_sec[pl.ds(k * CH + t * 16, 16)]
            for j in range(16):
                w = ea16[j]
                e = t * 16 + j
                for q in range(H // 16):
                    buf[e, pl.ds(q * 16, 16)] = (
                        buf[e, pl.ds(q * 16, 16)] * w)
            return carry2
        lax.fori_loop(0, CH // 16, grp, 0, unroll=2)

    def section(sec, carry):
        pltpu.sync_copy(srcg.at[s, sec], idx2d)
        pltpu.sync_copy(dstg.at[s, sec], dst2d)
        pltpu.sync_copy(ea.at[pl.ds(s * EP + sec * SECE, SECE)], ea_sec)

        def addcn(r, carry2):
            for q in range(CH // 16):
                idx2d[r, pl.ds(q * 16, 16)] = (
                    idx2d[r, pl.ds(q * 16, 16)] + cN)
            return carry2
        lax.fori_loop(0, SCH, addcn, 0)

        g_start(0, 0)
        g_start(1, 1)

        def triple(k3, carry2):
            k = 3 * k3
            g_wait(0)
            scale(k, 0)
            s_start(k, 0)

            @pl.when(k3 > 0)
            def _():
                s_wait(2)
            g_start(k + 2, 2)
            g_wait(1)
            scale(k + 1, 1)
            s_start(k + 1, 1)
            s_wait(0)
            g_start(k + 3, 0)
            g_wait(2)
            scale(k + 2, 2)
            s_start(k + 2, 2)
            s_wait(1)

            @pl.when(k3 < (SCH - 1) // 3 - 1)
            def _():
                g_start(k + 4, 1)
            return carry2

        lax.fori_loop(0, (SCH - 1) // 3, triple, 0)

        g_wait(0)
        scale(SCH - 1, 0)
        s_start(SCH - 1, 0)
        s_wait(2)
        s_wait(0)
        return carry

    lax.fori_loop(0, NSEC, section, 0)

    plsc.subcore_barrier()

    @pl.when(s < NS - 1)
    def _():
        pltpu.sync_copy(acc.at[pl.ds(n0, NZ)],
                        agg2.at[pl.ds(cN + n0, NZ)])

    @pl.when(s == NS - 1)
    def _():
        pltpu.sync_copy(acc.at[pl.ds(n0, NZL)],
                        agg2.at[pl.ds(cN + n0, NZL)])


_mp = pl.kernel(
    _mp_body,
    out_type=jax.ShapeDtypeStruct((NC * N, H), jnp.float32),
    mesh=plsc.VectorSubcoreMesh(core_axis_name="c", subcore_axis_name="s"),
    scratch_types=[
        pltpu.VMEM((SCH, CH), jnp.int32),
        pltpu.VMEM((SCH, CH), jnp.int32),
        pltpu.VMEM((SECE,), jnp.float32),
        pltpu.VMEM((CH, H), jnp.float32),
        pltpu.VMEM((CH, H), jnp.float32),
        pltpu.VMEM((CH, H), jnp.float32),
        pltpu.VMEM_SHARED((N, H), jnp.float32),
        pltpu.SemaphoreType.DMA,
        pltpu.SemaphoreType.DMA,
        pltpu.SemaphoreType.DMA,
        pltpu.SemaphoreType.DMA,
        pltpu.SemaphoreType.DMA,
        pltpu.SemaphoreType.DMA,
    ],
)


def _proj_body(x_ref, w_ref, b_ref, out_ref):
    h = jnp.dot(x_ref[...], w_ref[...], preferred_element_type=jnp.float32)
    h = jnp.maximum(h + b_ref[...], 0.0)
    out_ref[0] = h[:, :H]
    out_ref[1] = h[:, H:]


_proj = pl.pallas_call(
    _proj_body,
    grid=(GRID,),
    in_specs=[
        pl.BlockSpec((BLK, D_IN), lambda i: (i, 0)),
        pl.BlockSpec((D_IN, D_H), lambda i: (0, 0)),
        pl.BlockSpec((1, D_H), lambda i: (0, 0)),
    ],
    out_specs=pl.BlockSpec((2, BLK, H), lambda i: (0, i, 0)),
    out_shape=jax.ShapeDtypeStruct((2, N, H), jnp.float32),
)


def _mm_body(a0_ref, a1_ref, wt_ref, wb_ref, b_ref, out_ref, st_ref):
    i = pl.program_id(0)
    o = (jnp.dot(a0_ref[0], wt_ref[...], preferred_element_type=jnp.float32)
         + jnp.dot(a1_ref[0], wb_ref[...], preferred_element_type=jnp.float32)
         + b_ref[...])
    out_ref[...] = o
    st = jnp.concatenate([jnp.sum(o, axis=0)[None],
                          jnp.sum(o * o, axis=0)[None]], axis=0)

    @pl.when(i == 0)
    def _():
        st_ref[...] = st

    @pl.when(i > 0)
    def _():
        st_ref[...] = st_ref[...] + st


_mm_stats = pl.pallas_call(
    _mm_body,
    grid=(GRID,),
    in_specs=[
        pl.BlockSpec((1, BLK, H), lambda i: (0, i, 0)),
        pl.BlockSpec((1, BLK, H), lambda i: (1, i, 0)),
        pl.BlockSpec((H, D_H), lambda i: (0, 0)),
        pl.BlockSpec((H, D_H), lambda i: (0, 0)),
        pl.BlockSpec((1, D_H), lambda i: (0, 0)),
    ],
    out_specs=[
        pl.BlockSpec((BLK, D_H), lambda i: (i, 0)),
        pl.BlockSpec((2, D_H), lambda i: (0, 0)),
    ],
    out_shape=[
        jax.ShapeDtypeStruct((N, D_H), jnp.float32),
        jax.ShapeDtypeStruct((2, D_H), jnp.float32),
    ],
)


def _nr_body(o_ref, st_ref, g_ref, be_ref, h_ref, hn_ref, sm_ref):
    i = pl.program_id(0)
    inv_cnt = 1.0 / (N * D_H)
    mu = jnp.sum(st_ref[0:1, :]) * inv_cnt
    ex2 = jnp.sum(st_ref[1:2, :]) * inv_cnt
    inv = lax.rsqrt(ex2 - mu * mu + EPS)
    nrm = (o_ref[...] - mu) * inv * g_ref[...] + be_ref[...]
    h0 = jnp.maximum(nrm[:, :H] + h_ref[0], 0.0)
    h1 = jnp.maximum(nrm[:, H:] + h_ref[1], 0.0)
    hn_ref[0] = h0
    hn_ref[1] = h1
    sm = jnp.concatenate([jnp.sum(h0, axis=0)[None],
                          jnp.sum(h1, axis=0)[None]], axis=0)

    @pl.when(i == 0)
    def _():
        sm_ref[...] = sm

    @pl.when(i > 0)
    def _():
        sm_ref[...] = sm_ref[...] + sm


_norm_resid = pl.pallas_call(
    _nr_body,
    grid=(GRID,),
    in_specs=[
        pl.BlockSpec((BLK, D_H), lambda i: (i, 0)),
        pl.BlockSpec((2, D_H), lambda i: (0, 0)),
        pl.BlockSpec((1, D_H), lambda i: (0, 0)),
        pl.BlockSpec((1, D_H), lambda i: (0, 0)),
        pl.BlockSpec((2, BLK, H), lambda i: (0, i, 0)),
    ],
    out_specs=[
        pl.BlockSpec((2, BLK, H), lambda i: (0, i, 0)),
        pl.BlockSpec((2, H), lambda i: (0, 0)),
    ],
    out_shape=[
        jax.ShapeDtypeStruct((2, N, H), jnp.float32),
        jax.ShapeDtypeStruct((2, H), jnp.float32),
    ],
)


def _pool_body(sm_ref, wp_ref, bp_ref, out_ref):
    p = jnp.concatenate([sm_ref[0:1, :], sm_ref[1:2, :]], axis=1) * (1.0 / N)
    p8 = jnp.broadcast_to(p, (8, D_H))
    o = jnp.dot(p8, wp_ref[...], preferred_element_type=jnp.float32)
    o = jnp.maximum(o + bp_ref[...], 0.0)
    out_ref[...] = o[0:1, :]


_pool = pl.pallas_call(
    _pool_body,
    in_specs=[
        pl.BlockSpec((2, H), lambda: (0, 0)),
        pl.BlockSpec((D_H, D_H), lambda: (0, 0)),
        pl.BlockSpec((1, D_H), lambda: (0, 0)),
    ],
    out_specs=pl.BlockSpec((1, D_H), lambda: (0, 0)),
    out_shape=jax.ShapeDtypeStruct((1, D_H), jnp.float32),
)


def kernel(x, edge_index, edge_attr, W_in, b_in, W0, b0, g0, be0,
           W1, b1, g1, be1, W2, b2, g2, be2, W_pool, b_pool):
    srcg = edge_index[0].astype(jnp.int32).reshape(NS, NSEC, SCH, CH)
    dstg = edge_index[1].astype(jnp.int32).reshape(NS, NSEC, SCH, CH)
    ea = edge_attr.astype(jnp.float32)

    h = _proj(x, W_in, b_in.reshape(1, D_H))
    sums = None
    for (W, b, g, be) in ((W0, b0, g0, be0), (W1, b1, g1, be1),
                          (W2, b2, g2, be2)):
        agg2 = _mp(h.reshape(NC * N, H), srcg, dstg, ea)
        agg = agg2.reshape(NC, N, H)
        out, stats = _mm_stats(agg, agg, W[:H, :], W[H:, :],
                               b.reshape(1, D_H))
        h, sums = _norm_resid(out, stats, g.reshape(1, D_H),
                              be.reshape(1, D_H), h)
    return _pool(sums, W_pool, b_pool.reshape(1, D_H))


# speedup vs baseline: 1.1603x; 1.0085x over previous
import jax
import jax.numpy as jnp
from jax import lax
from jax.experimental import pallas as pl
from jax.experimental.pallas import tpu as pltpu
from jax.experimental.pallas import tpu_sc as plsc

N = 10000
E = 160000
D_IN = 128
D_H = 256
H = 128
EPS = 1e-5
BLK = 1000
GRID = N // BLK

NS = 16
NC = 2
EP = E // NS
CH = 80
NSEC = 5
SECE = EP // NSEC
SCH = SECE // CH
NZ = 640
NZL = N - (NS - 1) * NZ
ZR = 40


def _mp_body(h2, srcg, dstg, ea, agg2, idx2d, dst2d, ea_sec, b0, b1, b2,
             acc, g0, g1, g2, s0, s1, s2):
    bufs = (b0, b1, b2)
    gsems = (g0, g1, g2)
    ssems = (s0, s1, s2)
    c = lax.axis_index("c")
    s = lax.axis_index("s")
    cN = c * N
    n0 = s * NZ

    zv = jnp.zeros((16,), jnp.float32)
    for r in range(ZR):
        for q in range(H // 16):
            bufs[0][r, pl.ds(q * 16, 16)] = zv

    @pl.when(s < NS - 1)
    def _():
        zds = [pltpu.async_copy(bufs[0].at[pl.ds(0, ZR)],
                                acc.at[pl.ds(n0 + i * ZR, ZR)], gsems[0])
               for i in range(NZ // ZR)]
        for d in zds:
            d.wait()

    @pl.when(s == NS - 1)
    def _():
        zds = [pltpu.async_copy(bufs[0].at[pl.ds(0, ZR)],
                                acc.at[pl.ds(n0 + i * ZR, ZR)], gsems[0])
               for i in range(NZL // ZR)]
        for d in zds:
            d.wait()

    plsc.subcore_barrier()

    def g_start(k, x):
        pltpu.async_copy(h2.at[idx2d.at[k]], bufs[x], gsems[x])

    def g_wait(x):
        pltpu.make_async_copy(h2.at[idx2d.at[0]], bufs[x], gsems[x]).wait()

    def s_start(k, x):
        pltpu.async_copy(bufs[x], acc.at[dst2d.at[k]], ssems[x], add=True)

    def s_wait(x):
        pltpu.make_async_copy(bufs[x], acc.at[dst2d.at[0]], ssems[x]).wait()

    def scale(k, x):
        buf = bufs[x]

        def grp(t, carry2):
            ea16 = ea_sec[pl.ds(k * CH + t * 16, 16)]
            for j in range(16):
                w = ea16[j]
                e = t * 16 + j
                for q in range(H // 16):
                    buf[e, pl.ds(q * 16, 16)] = (
                        buf[e, pl.ds(q * 16, 16)] * w)
            return carry2
        lax.fori_loop(0, CH // 16, grp, 0)

    def section(sec, carry):
        pltpu.sync_copy(srcg.at[s, sec], idx2d)
        pltpu.sync_copy(dstg.at[s, sec], dst2d)
        pltpu.sync_copy(ea.at[pl.ds(s * EP + sec * SECE, SECE)], ea_sec)

        def addcn(r, carry2):
            for q in range(CH // 16):
                idx2d[r, pl.ds(q * 16, 16)] = (
                    idx2d[r, pl.ds(q * 16, 16)] + cN)
            return carry2
        lax.fori_loop(0, SCH, addcn, 0)

        g_start(0, 0)
        g_start(1, 1)

        def triple(k3, carry2):
            k = 3 * k3
            g_wait(0)
            scale(k, 0)
            s_start(k, 0)

            @pl.when(k3 > 0)
            def _():
                s_wait(2)
            g_start(k + 2, 2)
            g_wait(1)
            scale(k + 1, 1)
            s_start(k + 1, 1)
            s_wait(0)
            g_start(k + 3, 0)
            g_wait(2)
            scale(k + 2, 2)
            s_start(k + 2, 2)
            s_wait(1)

            @pl.when(k3 < (SCH - 1) // 3 - 1)
            def _():
                g_start(k + 4, 1)
            return carry2

        lax.fori_loop(0, (SCH - 1) // 3, triple, 0)

        g_wait(0)
        scale(SCH - 1, 0)
        s_start(SCH - 1, 0)
        s_wait(2)
        s_wait(0)
        return carry

    lax.fori_loop(0, NSEC, section, 0)

    plsc.subcore_barrier()

    @pl.when(s < NS - 1)
    def _():
        pltpu.sync_copy(acc.at[pl.ds(n0, NZ)],
                        agg2.at[pl.ds(cN + n0, NZ)])

    @pl.when(s == NS - 1)
    def _():
        pltpu.sync_copy(acc.at[pl.ds(n0, NZL)],
                        agg2.at[pl.ds(cN + n0, NZL)])


_mp = pl.kernel(
    _mp_body,
    out_type=jax.ShapeDtypeStruct((NC * N, H), jnp.float32),
    mesh=plsc.VectorSubcoreMesh(core_axis_name="c", subcore_axis_name="s"),
    scratch_types=[
        pltpu.VMEM((SCH, CH), jnp.int32),
        pltpu.VMEM((SCH, CH), jnp.int32),
        pltpu.VMEM((SECE,), jnp.float32),
        pltpu.VMEM((CH, H), jnp.float32),
        pltpu.VMEM((CH, H), jnp.float32),
        pltpu.VMEM((CH, H), jnp.float32),
        pltpu.VMEM_SHARED((N, H), jnp.float32),
        pltpu.SemaphoreType.DMA,
        pltpu.SemaphoreType.DMA,
        pltpu.SemaphoreType.DMA,
        pltpu.SemaphoreType.DMA,
        pltpu.SemaphoreType.DMA,
        pltpu.SemaphoreType.DMA,
    ],
)


def _proj_body(x_ref, w_ref, b_ref, out_ref):
    h = jnp.dot(x_ref[...], w_ref[...], preferred_element_type=jnp.float32)
    h = jnp.maximum(h + b_ref[...], 0.0)
    out_ref[0] = h[:, :H]
    out_ref[1] = h[:, H:]


_proj = pl.pallas_call(
    _proj_body,
    grid=(GRID,),
    in_specs=[
        pl.BlockSpec((BLK, D_IN), lambda i: (i, 0)),
        pl.BlockSpec((D_IN, D_H), lambda i: (0, 0)),
        pl.BlockSpec((1, D_H), lambda i: (0, 0)),
    ],
    out_specs=pl.BlockSpec((2, BLK, H), lambda i: (0, i, 0)),
    out_shape=jax.ShapeDtypeStruct((2, N, H), jnp.float32),
)


def _mm_body(a0_ref, a1_ref, wt_ref, wb_ref, b_ref, out_ref, st_ref):
    i = pl.program_id(0)
    o = (jnp.dot(a0_ref[0], wt_ref[...], preferred_element_type=jnp.float32)
         + jnp.dot(a1_ref[0], wb_ref[...], preferred_element_type=jnp.float32)
         + b_ref[...])
    out_ref[...] = o
    st = jnp.concatenate([jnp.sum(o, axis=0)[None],
                          jnp.sum(o * o, axis=0)[None]], axis=0)

    @pl.when(i == 0)
    def _():
        st_ref[...] = st

    @pl.when(i > 0)
    def _():
        st_ref[...] = st_ref[...] + st


_mm_stats = pl.pallas_call(
    _mm_body,
    grid=(GRID,),
    in_specs=[
        pl.BlockSpec((1, BLK, H), lambda i: (0, i, 0)),
        pl.BlockSpec((1, BLK, H), lambda i: (1, i, 0)),
        pl.BlockSpec((H, D_H), lambda i: (0, 0)),
        pl.BlockSpec((H, D_H), lambda i: (0, 0)),
        pl.BlockSpec((1, D_H), lambda i: (0, 0)),
    ],
    out_specs=[
        pl.BlockSpec((BLK, D_H), lambda i: (i, 0)),
        pl.BlockSpec((2, D_H), lambda i: (0, 0)),
    ],
    out_shape=[
        jax.ShapeDtypeStruct((N, D_H), jnp.float32),
        jax.ShapeDtypeStruct((2, D_H), jnp.float32),
    ],
)


def _nr_body(o_ref, st_ref, g_ref, be_ref, h_ref, hn_ref, sm_ref):
    i = pl.program_id(0)
    inv_cnt = 1.0 / (N * D_H)
    mu = jnp.sum(st_ref[0:1, :]) * inv_cnt
    ex2 = jnp.sum(st_ref[1:2, :]) * inv_cnt
    inv = lax.rsqrt(ex2 - mu * mu + EPS)
    nrm = (o_ref[...] - mu) * inv * g_ref[...] + be_ref[...]
    h0 = jnp.maximum(nrm[:, :H] + h_ref[0], 0.0)
    h1 = jnp.maximum(nrm[:, H:] + h_ref[1], 0.0)
    hn_ref[0] = h0
    hn_ref[1] = h1
    sm = jnp.concatenate([jnp.sum(h0, axis=0)[None],
                          jnp.sum(h1, axis=0)[None]], axis=0)

    @pl.when(i == 0)
    def _():
        sm_ref[...] = sm

    @pl.when(i > 0)
    def _():
        sm_ref[...] = sm_ref[...] + sm


_norm_resid = pl.pallas_call(
    _nr_body,
    grid=(GRID,),
    in_specs=[
        pl.BlockSpec((BLK, D_H), lambda i: (i, 0)),
        pl.BlockSpec((2, D_H), lambda i: (0, 0)),
        pl.BlockSpec((1, D_H), lambda i: (0, 0)),
        pl.BlockSpec((1, D_H), lambda i: (0, 0)),
        pl.BlockSpec((2, BLK, H), lambda i: (0, i, 0)),
    ],
    out_specs=[
        pl.BlockSpec((2, BLK, H), lambda i: (0, i, 0)),
        pl.BlockSpec((2, H), lambda i: (0, 0)),
    ],
    out_shape=[
        jax.ShapeDtypeStruct((2, N, H), jnp.float32),
        jax.ShapeDtypeStruct((2, H), jnp.float32),
    ],
)


def _pool_body(sm_ref, wp_ref, bp_ref, out_ref):
    p = jnp.concatenate([sm_ref[0:1, :], sm_ref[1:2, :]], axis=1) * (1.0 / N)
    p8 = jnp.broadcast_to(p, (8, D_H))
    o = jnp.dot(p8, wp_ref[...], preferred_element_type=jnp.float32)
    o = jnp.maximum(o + bp_ref[...], 0.0)
    out_ref[...] = o[0:1, :]


_pool = pl.pallas_call(
    _pool_body,
    in_specs=[
        pl.BlockSpec((2, H), lambda: (0, 0)),
        pl.BlockSpec((D_H, D_H), lambda: (0, 0)),
        pl.BlockSpec((1, D_H), lambda: (0, 0)),
    ],
    out_specs=pl.BlockSpec((1, D_H), lambda: (0, 0)),
    out_shape=jax.ShapeDtypeStruct((1, D_H), jnp.float32),
)


def kernel(x, edge_index, edge_attr, W_in, b_in, W0, b0, g0, be0,
           W1, b1, g1, be1, W2, b2, g2, be2, W_pool, b_pool):
    srcg = edge_index[0].astype(jnp.int32).reshape(NS, NSEC, SCH, CH)
    dstg = edge_index[1].astype(jnp.int32).reshape(NS, NSEC, SCH, CH)
    ea = edge_attr.astype(jnp.float32)

    h = _proj(x, W_in, b_in.reshape(1, D_H))
    sums = None
    for (W, b, g, be) in ((W0, b0, g0, be0), (W1, b1, g1, be1),
                          (W2, b2, g2, be2)):
        agg2 = _mp(h.reshape(NC * N, H), srcg, dstg, ea)
        agg = agg2.reshape(NC, N, H)
        out, stats = _mm_stats(agg, agg, W[:H, :], W[H:, :],
                               b.reshape(1, D_H))
        h, sums = _norm_resid(out, stats, g.reshape(1, D_H),
                              be.reshape(1, D_H), h)
    return _pool(sums, W_pool, b_pool.reshape(1, D_H))


# speedup vs baseline: 1.1845x; 1.0208x over previous
import jax
import jax.numpy as jnp
from jax import lax
from jax.experimental import pallas as pl
from jax.experimental.pallas import tpu as pltpu
from jax.experimental.pallas import tpu_sc as plsc

N = 10000
E = 160000
D_IN = 128
D_H = 256
H = 128
EPS = 1e-5
BLK = 1000
GRID = N // BLK

NS = 16
NC = 2
EP = E // NS
CH = 80
NSEC = 5
SECE = EP // NSEC
SCH = SECE // CH
NZ = 640
NZL = N - (NS - 1) * NZ
ZR = 40


def _mp_body(h2, srcg, dstg, ea, agg2, idx2d, dst2d, ea_sec, b0, b1, b2,
             acc, g0, g1, g2, s0, s1, s2):
    bufs = (b0, b1, b2)
    gsems = (g0, g1, g2)
    ssems = (s0, s1, s2)
    c = lax.axis_index("c")
    s = lax.axis_index("s")
    cN = c * N
    n0 = s * NZ

    zv = jnp.zeros((16,), jnp.float32)
    for r in range(ZR):
        for q in range(H // 16):
            bufs[0][r, pl.ds(q * 16, 16)] = zv

    @pl.when(s < NS - 1)
    def _():
        zds = [pltpu.async_copy(bufs[0].at[pl.ds(0, ZR)],
                                acc.at[pl.ds(n0 + i * ZR, ZR)], gsems[0])
               for i in range(NZ // ZR)]
        for d in zds:
            d.wait()

    @pl.when(s == NS - 1)
    def _():
        zds = [pltpu.async_copy(bufs[0].at[pl.ds(0, ZR)],
                                acc.at[pl.ds(n0 + i * ZR, ZR)], gsems[0])
               for i in range(NZL // ZR)]
        for d in zds:
            d.wait()

    plsc.subcore_barrier()

    def g_start(k, x):
        pltpu.async_copy(h2.at[idx2d.at[k]], bufs[x], gsems[x])

    def g_wait(x):
        pltpu.make_async_copy(h2.at[idx2d.at[0]], bufs[x], gsems[x]).wait()

    def s_start(k, x):
        pltpu.async_copy(bufs[x], acc.at[dst2d.at[k]], ssems[x], add=True)

    def s_wait(x):
        pltpu.make_async_copy(bufs[x], acc.at[dst2d.at[0]], ssems[x]).wait()

    def scale(k, x):
        buf = bufs[x]

        def grp(t, carry2):
            ea16 = ea_sec[pl.ds(k * CH + t * 16, 16)]
            for j in range(16):
                w = ea16[j]
                e = t * 16 + j
                for q in range(H // 16):
                    buf[e, pl.ds(q * 16, 16)] = (
                        buf[e, pl.ds(q * 16, 16)] * w)
            return carry2
        lax.fori_loop(0, CH // 16, grp, 0)

    def section(sec, carry):
        pltpu.sync_copy(srcg.at[s, sec], idx2d)
        pltpu.sync_copy(dstg.at[s, sec], dst2d)
        pltpu.sync_copy(ea.at[pl.ds(s * EP + sec * SECE, SECE)], ea_sec)

        def addcn(r, carry2):
            for q in range(CH // 16):
                idx2d[r, pl.ds(q * 16, 16)] = (
                    idx2d[r, pl.ds(q * 16, 16)] + cN)
            return carry2
        lax.fori_loop(0, SCH, addcn, 0)

        g_start(0, 0)
        g_start(1, 1)

        def triple(k3, carry2):
            k = 3 * k3
            g_wait(0)
            scale(k, 0)
            s_start(k, 0)

            @pl.when(k3 > 0)
            def _():
                s_wait(2)
            g_start(k + 2, 2)
            g_wait(1)
            scale(k + 1, 1)
            s_start(k + 1, 1)
            s_wait(0)
            g_start(k + 3, 0)
            g_wait(2)
            scale(k + 2, 2)
            s_start(k + 2, 2)
            s_wait(1)

            @pl.when(k3 < (SCH - 1) // 3 - 1)
            def _():
                g_start(k + 4, 1)
            return carry2

        lax.fori_loop(0, (SCH - 1) // 3, triple, 0)

        g_wait(0)
        scale(SCH - 1, 0)
        s_start(SCH - 1, 0)
        s_wait(2)
        s_wait(0)
        return carry

    lax.fori_loop(0, NSEC, section, 0)

    plsc.subcore_barrier()

    @pl.when(s < NS - 1)
    def _():
        pltpu.sync_copy(acc.at[pl.ds(n0, NZ)],
                        agg2.at[pl.ds(cN + n0, NZ)])

    @pl.when(s == NS - 1)
    def _():
        pltpu.sync_copy(acc.at[pl.ds(n0, NZL)],
                        agg2.at[pl.ds(cN + n0, NZL)])


_mp = pl.kernel(
    _mp_body,
    out_type=jax.ShapeDtypeStruct((NC * N, H), jnp.float32),
    mesh=plsc.VectorSubcoreMesh(core_axis_name="c", subcore_axis_name="s"),
    scratch_types=[
        pltpu.VMEM((SCH, CH), jnp.int32),
        pltpu.VMEM((SCH, CH), jnp.int32),
        pltpu.VMEM((SECE,), jnp.float32),
        pltpu.VMEM((CH, H), jnp.float32),
        pltpu.VMEM((CH, H), jnp.float32),
        pltpu.VMEM((CH, H), jnp.float32),
        pltpu.VMEM_SHARED((N, H), jnp.float32),
        pltpu.SemaphoreType.DMA,
        pltpu.SemaphoreType.DMA,
        pltpu.SemaphoreType.DMA,
        pltpu.SemaphoreType.DMA,
        pltpu.SemaphoreType.DMA,
        pltpu.SemaphoreType.DMA,
    ],
)


def _proj_body(x_ref, w_ref, b_ref, out_ref):
    h = jnp.dot(x_ref[...], w_ref[...], preferred_element_type=jnp.float32)
    h = jnp.maximum(h + b_ref[...], 0.0)
    out_ref[0] = h[:, :H]
    out_ref[1] = h[:, H:]


_proj = pl.pallas_call(
    _proj_body,
    grid=(GRID,),
    in_specs=[
        pl.BlockSpec((BLK, D_IN), lambda i: (i, 0)),
        pl.BlockSpec((D_IN, D_H), lambda i: (0, 0)),
        pl.BlockSpec((1, D_H), lambda i: (0, 0)),
    ],
    out_specs=pl.BlockSpec((2, BLK, H), lambda i: (0, i, 0)),
    out_shape=jax.ShapeDtypeStruct((2, N, H), jnp.float32),
)


def _layer_body(a0_ref, a1_ref, wt_ref, wb_ref, b_ref, g_ref, be_ref,
                h_ref, hn_ref, sm_ref, ob_ref, st_ref):
    p = pl.program_id(0)
    i = pl.program_id(1)

    @pl.when(p == 0)
    def _():
        o = (jnp.dot(a0_ref[0], wt_ref[...],
                     preferred_element_type=jnp.float32)
             + jnp.dot(a1_ref[0], wb_ref[...],
                       preferred_element_type=jnp.float32)
             + b_ref[...])
        ob_ref[pl.ds(i * BLK, BLK), :] = o
        st = jnp.concatenate([jnp.sum(o, axis=0)[None],
                              jnp.sum(o * o, axis=0)[None]], axis=0)

        @pl.when(i == 0)
        def _():
            st_ref[...] = st

        @pl.when(i > 0)
        def _():
            st_ref[...] = st_ref[...] + st

    @pl.when(p == 1)
    def _():
        o = ob_ref[pl.ds(i * BLK, BLK), :]
        inv_cnt = 1.0 / (N * D_H)
        mu = jnp.sum(st_ref[0:1, :]) * inv_cnt
        ex2 = jnp.sum(st_ref[1:2, :]) * inv_cnt
        inv = lax.rsqrt(ex2 - mu * mu + EPS)
        nrm = (o - mu) * inv * g_ref[...] + be_ref[...]
        h0 = jnp.maximum(nrm[:, :H] + h_ref[0], 0.0)
        h1 = jnp.maximum(nrm[:, H:] + h_ref[1], 0.0)
        hn_ref[0] = h0
        hn_ref[1] = h1
        sm = jnp.concatenate([jnp.sum(h0, axis=0)[None],
                              jnp.sum(h1, axis=0)[None]], axis=0)

        @pl.when(i == 0)
        def _():
            sm_ref[...] = sm

        @pl.when(i > 0)
        def _():
            sm_ref[...] = sm_ref[...] + sm


_layer = pl.pallas_call(
    _layer_body,
    grid=(2, GRID),
    in_specs=[
        pl.BlockSpec((1, BLK, H), lambda p, i: (0, i * (1 - p), 0)),
        pl.BlockSpec((1, BLK, H), lambda p, i: (1, i * (1 - p), 0)),
        pl.BlockSpec((H, D_H), lambda p, i: (0, 0)),
        pl.BlockSpec((H, D_H), lambda p, i: (0, 0)),
        pl.BlockSpec((1, D_H), lambda p, i: (0, 0)),
        pl.BlockSpec((1, D_H), lambda p, i: (0, 0)),
        pl.BlockSpec((1, D_H), lambda p, i: (0, 0)),
        pl.BlockSpec((2, BLK, H), lambda p, i: (0, i * p, 0)),
    ],
    out_specs=[
        pl.BlockSpec((2, BLK, H), lambda p, i: (0, i, 0)),
        pl.BlockSpec((2, H), lambda p, i: (0, 0)),
    ],
    out_shape=[
        jax.ShapeDtypeStruct((2, N, H), jnp.float32),
        jax.ShapeDtypeStruct((2, H), jnp.float32),
    ],
    scratch_shapes=[
        pltpu.VMEM((N, D_H), jnp.float32),
        pltpu.VMEM((2, D_H), jnp.float32),
    ],
)


def _pool_body(sm_ref, wp_ref, bp_ref, out_ref):
    p = jnp.concatenate([sm_ref[0:1, :], sm_ref[1:2, :]], axis=1) * (1.0 / N)
    p8 = jnp.broadcast_to(p, (8, D_H))
    o = jnp.dot(p8, wp_ref[...], preferred_element_type=jnp.float32)
    o = jnp.maximum(o + bp_ref[...], 0.0)
    out_ref[...] = o[0:1, :]


_pool = pl.pallas_call(
    _pool_body,
    in_specs=[
        pl.BlockSpec((2, H), lambda: (0, 0)),
        pl.BlockSpec((D_H, D_H), lambda: (0, 0)),
        pl.BlockSpec((1, D_H), lambda: (0, 0)),
    ],
    out_specs=pl.BlockSpec((1, D_H), lambda: (0, 0)),
    out_shape=jax.ShapeDtypeStruct((1, D_H), jnp.float32),
)


def kernel(x, edge_index, edge_attr, W_in, b_in, W0, b0, g0, be0,
           W1, b1, g1, be1, W2, b2, g2, be2, W_pool, b_pool):
    srcg = edge_index[0].astype(jnp.int32).reshape(NS, NSEC, SCH, CH)
    dstg = edge_index[1].astype(jnp.int32).reshape(NS, NSEC, SCH, CH)
    ea = edge_attr.astype(jnp.float32)

    h = _proj(x, W_in, b_in.reshape(1, D_H))
    sums = None
    for (W, b, g, be) in ((W0, b0, g0, be0), (W1, b1, g1, be1),
                          (W2, b2, g2, be2)):
        agg2 = _mp(h.reshape(NC * N, H), srcg, dstg, ea)
        agg = agg2.reshape(NC, N, H)
        h, sums = _layer(agg, agg, W[:H, :], W[H:, :], b.reshape(1, D_H),
                         g.reshape(1, D_H), be.reshape(1, D_H), h)
    return _pool(sums, W_pool, b_pool.reshape(1, D_H))
